# baseline (speedup 1.0000x reference)
import jax
import jax.numpy as jnp
from jax.experimental import pallas as pl

_S_BLK = 1024


def _bind_kernel(x_ref, p_ref, o_ref):
    o_ref[...] = x_ref[...] * p_ref[...]


def kernel(input_tensor, position_vectors):
    bsz, seq_len, d = input_tensor.shape
    pos = position_vectors[:seq_len, :d]
    s_blk = 256
    grid = (seq_len // s_blk,)
    return pl.pallas_call(
        _bind_kernel,
        grid=grid,
        in_specs=[
            pl.BlockSpec((bsz, s_blk, d), lambda s: (0, s, 0)),
            pl.BlockSpec((s_blk, d), lambda s: (s, 0)),
        ],
        out_specs=pl.BlockSpec((bsz, s_blk, d), lambda s: (0, s, 0)),
        out_shape=jax.ShapeDtypeStruct((bsz, seq_len, d), input_tensor.dtype),
    )(input_tensor, pos)
